# Initial kernel scaffold; baseline (speedup 1.0000x reference)
import jax
import jax.numpy as jnp
from jax.experimental import pallas as pl


def kernel(prev_out, prev_out2, curr_h, W):
    raise NotImplementedError("write your pallas kernel here")


# speedup vs baseline: 1.3874x; 1.3874x over previous
import functools

import jax
import jax.numpy as jnp
from jax.experimental import pallas as pl

_THR = 0.999
_N = 16384
_D = 1024
_R = 512


def _body(a_ref, b_ref, c_ref, wd_ref, out_ref, es_ref):
    a = a_ref[...]
    b = b_ref[...]
    c = c_ref[...]
    wd = wd_ref[...]
    t1 = jnp.sum(a * wd, axis=1, keepdims=True)
    t2 = jnp.sum(b * wd, axis=1, keepdims=True)
    g1 = 1.0 / (1.0 + jnp.exp(-t1))
    g2p = 1.0 / (1.0 + jnp.exp(-t2))
    g2 = jnp.where(g1 >= _THR, 0.0, (1.0 - g1) * g2p)
    acc = g1 + g2
    rem = 1.0 - acc
    coef_c = jnp.where(acc >= _THR, 0.0, rem)
    out_ref[...] = g1 * a + g2 * b + coef_c * c
    es_ref[...] = g2 + 2.0 * rem


@jax.jit
def kernel(prev_out, prev_out2, curr_h, W):
    wd = (W[1] - W[0]).reshape(1, _D)
    grid = (_N // _R,)
    row_spec = pl.BlockSpec((_R, _D), lambda i: (i, 0))
    out, es = pl.pallas_call(
        _body,
        grid=grid,
        in_specs=[
            row_spec,
            row_spec,
            row_spec,
            pl.BlockSpec((1, _D), lambda i: (0, 0)),
        ],
        out_specs=[
            row_spec,
            pl.BlockSpec((_R, 1), lambda i: (i, 0)),
        ],
        out_shape=[
            jax.ShapeDtypeStruct((_N, _D), jnp.float32),
            jax.ShapeDtypeStruct((_N, 1), jnp.float32),
        ],
    )(prev_out, prev_out2, curr_h, wd)
    return out, es.reshape(_N)


# speedup vs baseline: 1.3905x; 1.0023x over previous
import functools

import jax
import jax.numpy as jnp
from jax.experimental import pallas as pl

_THR = 0.999
_N = 16384
_D = 1024
_R = 1024


def _body(a_ref, b_ref, c_ref, wd_ref, out_ref, es_ref):
    a = a_ref[...]
    b = b_ref[...]
    c = c_ref[...]
    wd = wd_ref[...]
    t1 = jnp.sum(a * wd, axis=1, keepdims=True)
    t2 = jnp.sum(b * wd, axis=1, keepdims=True)
    g1 = 1.0 / (1.0 + jnp.exp(-t1))
    g2p = 1.0 / (1.0 + jnp.exp(-t2))
    g2 = jnp.where(g1 >= _THR, 0.0, (1.0 - g1) * g2p)
    acc = g1 + g2
    rem = 1.0 - acc
    coef_c = jnp.where(acc >= _THR, 0.0, rem)
    out_ref[...] = g1 * a + g2 * b + coef_c * c
    es_ref[...] = g2 + 2.0 * rem


@jax.jit
def kernel(prev_out, prev_out2, curr_h, W):
    wd = (W[1] - W[0]).reshape(1, _D)
    grid = (_N // _R,)
    row_spec = pl.BlockSpec((_R, _D), lambda i: (i, 0))
    out, es = pl.pallas_call(
        _body,
        grid=grid,
        in_specs=[
            row_spec,
            row_spec,
            row_spec,
            pl.BlockSpec((1, _D), lambda i: (0, 0)),
        ],
        out_specs=[
            row_spec,
            pl.BlockSpec((_R, 1), lambda i: (i, 0)),
        ],
        out_shape=[
            jax.ShapeDtypeStruct((_N, _D), jnp.float32),
            jax.ShapeDtypeStruct((_N, 1), jnp.float32),
        ],
    )(prev_out, prev_out2, curr_h, wd)
    return out, es.reshape(_N)
